# Initial kernel scaffold; baseline (speedup 1.0000x reference)
import functools

import jax
import jax.numpy as jnp
from jax import lax
from jax.experimental import pallas as pl
from jax.experimental.pallas import tpu as pltpu
from jax.experimental.pallas import tpu_sc as plsc

_N = 10000
_T = 20000
_E = 320000
_C = 128
_OUT = 10

_NC = 2
_NS = 16
_NW = _NC * _NS

_NP = 10240
_ROWS_SUB = _NP // _NS
_XPW = _NP // _NW
_XCH = 80
_ECH = 128
_EPW = 10112
_EP = _EPW * _NW
_DW = 16

_BLK = 1024
_GRID = _NP // _BLK

_mesh = plsc.VectorSubcoreMesh(core_axis_name="c", subcore_axis_name="s")


@functools.partial(
    pl.kernel,
    out_type=jax.ShapeDtypeStruct((_NP, _C), jnp.float32),
    mesh=_mesh,
    scratch_types=[
        pltpu.VMEM((_XCH,), jnp.int32),
        pltpu.VMEM((_XCH, _C), jnp.float32),
        pltpu.SemaphoreType.DMA,
    ],
)
def _sc_gather_x(tbl_hbm, nidx_hbm, x_hbm, idx_v, rows_v, sem):
    wid = lax.axis_index("s") * _NC + lax.axis_index("c")
    base = wid * _XPW
    for j in range(_XPW // _XCH):
        off = base + j * _XCH
        pltpu.sync_copy(nidx_hbm.at[pl.ds(off, _XCH)], idx_v)
        pltpu.async_copy(tbl_hbm.at[idx_v], rows_v, sem).wait()
        pltpu.sync_copy(rows_v, x_hbm.at[pl.ds(off, _XCH)])


def _make_agg(with_deg):
    out_types = [jax.ShapeDtypeStruct((_NC * _NP, _C), jnp.float32)]
    scratch = [
        pltpu.VMEM((_ECH,), jnp.int32),
        pltpu.VMEM((_ECH,), jnp.int32),
        pltpu.VMEM((_ECH, _C), jnp.float32),
        pltpu.VMEM((_ECH, _C), jnp.float32),
        pltpu.VMEM_SHARED((_NP, _C), jnp.float32),
        pltpu.SemaphoreType.DMA,
    ]
    if with_deg:
        out_types.append(jax.ShapeDtypeStruct((_NC * _NP, _DW), jnp.float32))
        scratch += [
            pltpu.VMEM((_ECH, _DW), jnp.float32),
            pltpu.VMEM((_ECH, _DW), jnp.float32),
            pltpu.VMEM_SHARED((_NP, _DW), jnp.float32),
        ]

    def body(src_hbm, dst_hbm, vals_hbm, *rest):
        if with_deg:
            (acc_hbm, deg_hbm, sidx, didx, rows_v, zbuf, acc, sem,
             ones_v, z16, dacc) = rest
        else:
            acc_hbm, sidx, didx, rows_v, zbuf, acc, sem = rest
        cid = lax.axis_index("c")
        sid = lax.axis_index("s")
        wid = sid * _NC + cid

        z = jnp.zeros((16,), jnp.float32)
        o = jnp.ones((16,), jnp.float32)

        @pl.loop(0, _ECH)
        def _(r):
            for j in range(_C // 16):
                zbuf[r, pl.ds(j * 16, 16)] = z
            if with_deg:
                ones_v[r, pl.ds(0, _DW)] = o
                z16[r, pl.ds(0, _DW)] = z

        rbase = sid * _ROWS_SUB
        for j in range(_ROWS_SUB // _ECH):
            pltpu.sync_copy(zbuf, acc.at[pl.ds(rbase + j * _ECH, _ECH)])
            if with_deg:
                pltpu.sync_copy(z16, dacc.at[pl.ds(rbase + j * _ECH, _ECH)])
        plsc.subcore_barrier()

        ebase = wid * _EPW

        @pl.loop(0, _EPW // _ECH)
        def _(ci):
            off = ebase + ci * _ECH
            pltpu.sync_copy(src_hbm.at[pl.ds(off, _ECH)], sidx)
            pltpu.sync_copy(dst_hbm.at[pl.ds(off, _ECH)], didx)
            pltpu.async_copy(vals_hbm.at[sidx], rows_v, sem).wait()
            pltpu.sync_copy(rows_v, acc.at[didx], add=True)
            if with_deg:
                pltpu.sync_copy(ones_v, dacc.at[didx], add=True)

        plsc.subcore_barrier()
        obase = cid * _NP + rbase
        for j in range(_ROWS_SUB // _ECH):
            pltpu.sync_copy(acc.at[pl.ds(rbase + j * _ECH, _ECH)],
                            acc_hbm.at[pl.ds(obase + j * _ECH, _ECH)])
            if with_deg:
                pltpu.sync_copy(dacc.at[pl.ds(rbase + j * _ECH, _ECH)],
                                deg_hbm.at[pl.ds(obase + j * _ECH, _ECH)])

    return pl.kernel(body, out_type=out_types, mesh=_mesh,
                     scratch_types=scratch)


_sc_agg_deg = _make_agg(True)
_sc_agg = _make_agg(False)


def _tc1_body(x_ref, p0_ref, p1_ref, d0_ref, d1_ref, wr_ref, wn_ref, b_ref,
              h_ref):
    deg = jnp.maximum(d0_ref[:, :1] + d1_ref[:, :1], 1.0)
    agg = (p0_ref[...] + p1_ref[...]) / deg
    h = (jnp.dot(x_ref[...], wr_ref[...], preferred_element_type=jnp.float32)
         + jnp.dot(agg, wn_ref[...], preferred_element_type=jnp.float32)
         + b_ref[...])
    h_ref[...] = jnp.maximum(h, 0.0)


def _tc2_body(h_ref, p0_ref, p1_ref, d0_ref, d1_ref, wr_ref, wn_ref, b_ref,
              wh_ref, bh_ref, out_ref):
    deg = jnp.maximum(d0_ref[:, :1] + d1_ref[:, :1], 1.0)
    agg = (p0_ref[...] + p1_ref[...]) / deg
    h2 = (jnp.dot(h_ref[...], wr_ref[...], preferred_element_type=jnp.float32)
          + jnp.dot(agg, wn_ref[...], preferred_element_type=jnp.float32)
          + b_ref[...])
    out_ref[...] = (jnp.dot(h2, wh_ref[...],
                            preferred_element_type=jnp.float32) + bh_ref[...])


def _row_specs():
    return [
        pl.BlockSpec((_BLK, _C), lambda i: (i, 0)),
        pl.BlockSpec((_BLK, _C), lambda i: (i, 0)),
        pl.BlockSpec((_BLK, _C), lambda i: (i + _GRID, 0)),
        pl.BlockSpec((_BLK, _DW), lambda i: (i, 0)),
        pl.BlockSpec((_BLK, _DW), lambda i: (i + _GRID, 0)),
        pl.BlockSpec((_C, _C), lambda i: (0, 0)),
        pl.BlockSpec((_C, _C), lambda i: (0, 0)),
        pl.BlockSpec((1, _C), lambda i: (0, 0)),
    ]


_tc1 = pl.pallas_call(
    _tc1_body,
    out_shape=jax.ShapeDtypeStruct((_NP, _C), jnp.float32),
    grid=(_GRID,),
    in_specs=_row_specs(),
    out_specs=pl.BlockSpec((_BLK, _C), lambda i: (i, 0)),
)

_tc2 = pl.pallas_call(
    _tc2_body,
    out_shape=jax.ShapeDtypeStruct((_NP, _OUT), jnp.float32),
    grid=(_GRID,),
    in_specs=_row_specs() + [
        pl.BlockSpec((_C, _OUT), lambda i: (0, 0)),
        pl.BlockSpec((1, _OUT), lambda i: (0, 0)),
    ],
    out_specs=pl.BlockSpec((_BLK, _OUT), lambda i: (i, 0)),
)


def kernel(feat_table, node_idx, edge_index,
           W_root1, W_nbr1, b1, W_root2, W_nbr2, b2, W_head, b_head):
    nidx = jnp.concatenate(
        [node_idx, jnp.zeros((_NP - _N,), jnp.int32)])
    src = jnp.concatenate(
        [edge_index[0], jnp.zeros((_EP - _E,), jnp.int32)])
    dst = jnp.concatenate(
        [edge_index[1], jnp.full((_EP - _E,), _N, jnp.int32)])

    x = _sc_gather_x(feat_table, nidx)
    acc1, deg = _sc_agg_deg(src, dst, x)
    h = _tc1(x, acc1, acc1, deg, deg, W_root1, W_nbr1, b1.reshape(1, _C))
    acc2 = _sc_agg(src, dst, h)
    out = _tc2(h, acc2, acc2, deg, deg, W_root2, W_nbr2, b2.reshape(1, _C),
               W_head, b_head.reshape(1, _OUT))
    return out[:_N]


# speedup vs baseline: 4.1872x; 4.1872x over previous
import functools

import jax
import jax.numpy as jnp
from jax import lax
from jax.experimental import pallas as pl
from jax.experimental.pallas import tpu as pltpu
from jax.experimental.pallas import tpu_sc as plsc

_N = 10000
_T = 20000
_E = 320000
_C = 128
_OUT = 10

_NC = 2
_NS = 16
_NW = _NC * _NS

_NP = 10240
_ROWS_SUB = _NP // _NS
_XPW = _NP // _NW
_XCH = 80
_ECH = 128
_EPW = 10112
_EP = _EPW * _NW
_DW = 128
_ZR = 64

_BLK = 1024
_GRID = _NP // _BLK


@functools.cache
def _get_mesh():
    return plsc.VectorSubcoreMesh(core_axis_name="c", subcore_axis_name="s",
                                  num_cores=_NC, num_subcores=_NS)


@functools.cache
def _get_gather_x_deg():
    @functools.partial(
        pl.kernel,
        out_type=[
            jax.ShapeDtypeStruct((_NP, _C), jnp.float32),
            jax.ShapeDtypeStruct((_NC * _NP, _DW), jnp.float32),
        ],
        mesh=_get_mesh(),
        scratch_types=[
            pltpu.VMEM((_XCH,), jnp.int32),
            pltpu.VMEM((_XCH, _C), jnp.float32),
            pltpu.VMEM((_ECH,), jnp.int32),
            pltpu.VMEM((_ECH, _DW), jnp.float32),
            pltpu.VMEM((_ZR, _DW), jnp.float32),
            pltpu.VMEM_SHARED((_NP, _DW), jnp.float32),
            pltpu.SemaphoreType.DMA,
        ],
    )
    def _sc_gather_x_deg(tbl_hbm, nidx_hbm, dst_hbm, x_hbm, deg_hbm,
                         idx_v, rows_v, didx, ones_v, z16, dacc, sem):
        cid = lax.axis_index("c")
        sid = lax.axis_index("s")
        wid = sid * _NC + cid

        z = jnp.zeros((16,), jnp.float32)
        o = jnp.ones((16,), jnp.float32)

        @pl.loop(0, _ECH)
        def _(r):
            for j in range(_DW // 16):
                ones_v[r, pl.ds(j * 16, 16)] = o

        @pl.loop(0, _ZR)
        def _(r):
            for j in range(_DW // 16):
                z16[r, pl.ds(j * 16, 16)] = z

        rbase = sid * _ROWS_SUB
        for j in range(_ROWS_SUB // _ZR):
            pltpu.sync_copy(z16, dacc.at[pl.ds(rbase + j * _ZR, _ZR)])
        plsc.subcore_barrier()

        base = wid * _XPW
        for j in range(_XPW // _XCH):
            off = base + j * _XCH
            pltpu.sync_copy(nidx_hbm.at[pl.ds(off, _XCH)], idx_v)
            pltpu.async_copy(tbl_hbm.at[idx_v], rows_v, sem).wait()
            pltpu.sync_copy(rows_v, x_hbm.at[pl.ds(off, _XCH)])

        ebase = wid * _EPW

        @pl.loop(0, _EPW // _ECH)
        def _(ci):
            off = ebase + ci * _ECH
            pltpu.sync_copy(dst_hbm.at[pl.ds(off, _ECH)], didx)
            pltpu.sync_copy(ones_v, dacc.at[didx], add=True)

        plsc.subcore_barrier()
        obase = cid * _NP + rbase
        for j in range(_ROWS_SUB // _ECH):
            pltpu.sync_copy(dacc.at[pl.ds(rbase + j * _ECH, _ECH)],
                            deg_hbm.at[pl.ds(obase + j * _ECH, _ECH)])

    return _sc_gather_x_deg


@functools.cache
def _get_agg():
    @functools.partial(
        pl.kernel,
        out_type=jax.ShapeDtypeStruct((_NC * _NP, _C), jnp.float32),
        mesh=_get_mesh(),
        scratch_types=[
            pltpu.VMEM((_ECH,), jnp.int32),
            pltpu.VMEM((_ECH,), jnp.int32),
            pltpu.VMEM((_ECH, _C), jnp.float32),
            pltpu.VMEM((_ZR, _C), jnp.float32),
            pltpu.VMEM_SHARED((_NP, _C), jnp.float32),
            pltpu.SemaphoreType.DMA,
        ],
    )
    def _sc_agg(src_hbm, dst_hbm, vals_hbm, acc_hbm,
                sidx, didx, rows_v, zbuf, acc, sem):
        cid = lax.axis_index("c")
        sid = lax.axis_index("s")
        wid = sid * _NC + cid

        z = jnp.zeros((16,), jnp.float32)

        @pl.loop(0, _ZR)
        def _(r):
            for j in range(_C // 16):
                zbuf[r, pl.ds(j * 16, 16)] = z

        rbase = sid * _ROWS_SUB
        for j in range(_ROWS_SUB // _ZR):
            pltpu.sync_copy(zbuf, acc.at[pl.ds(rbase + j * _ZR, _ZR)])
        plsc.subcore_barrier()

        ebase = wid * _EPW

        @pl.loop(0, _EPW // _ECH)
        def _(ci):
            off = ebase + ci * _ECH
            pltpu.sync_copy(src_hbm.at[pl.ds(off, _ECH)], sidx)
            pltpu.sync_copy(dst_hbm.at[pl.ds(off, _ECH)], didx)
            pltpu.async_copy(vals_hbm.at[sidx], rows_v, sem).wait()
            pltpu.sync_copy(rows_v, acc.at[didx], add=True)

        plsc.subcore_barrier()
        obase = cid * _NP + rbase
        for j in range(_ROWS_SUB // _ECH):
            pltpu.sync_copy(acc.at[pl.ds(rbase + j * _ECH, _ECH)],
                            acc_hbm.at[pl.ds(obase + j * _ECH, _ECH)])

    return _sc_agg


def _tc1_body(x_ref, p0_ref, p1_ref, d0_ref, d1_ref, wr_ref, wn_ref, b_ref,
              h_ref):
    deg = jnp.maximum(d0_ref[:, :1] + d1_ref[:, :1], 1.0)
    agg = (p0_ref[...] + p1_ref[...]) / deg
    h = (jnp.dot(x_ref[...], wr_ref[...], preferred_element_type=jnp.float32)
         + jnp.dot(agg, wn_ref[...], preferred_element_type=jnp.float32)
         + b_ref[...])
    h_ref[...] = jnp.maximum(h, 0.0)


def _tc2_body(h_ref, p0_ref, p1_ref, d0_ref, d1_ref, wr_ref, wn_ref, b_ref,
              wh_ref, bh_ref, out_ref):
    deg = jnp.maximum(d0_ref[:, :1] + d1_ref[:, :1], 1.0)
    agg = (p0_ref[...] + p1_ref[...]) / deg
    h2 = (jnp.dot(h_ref[...], wr_ref[...], preferred_element_type=jnp.float32)
          + jnp.dot(agg, wn_ref[...], preferred_element_type=jnp.float32)
          + b_ref[...])
    out_ref[...] = (jnp.dot(h2, wh_ref[...],
                            preferred_element_type=jnp.float32) + bh_ref[...])


def _row_specs():
    return [
        pl.BlockSpec((_BLK, _C), lambda i: (i, 0)),
        pl.BlockSpec((_BLK, _C), lambda i: (i, 0)),
        pl.BlockSpec((_BLK, _C), lambda i: (i + _GRID, 0)),
        pl.BlockSpec((_BLK, _DW), lambda i: (i, 0)),
        pl.BlockSpec((_BLK, _DW), lambda i: (i + _GRID, 0)),
        pl.BlockSpec((_C, _C), lambda i: (0, 0)),
        pl.BlockSpec((_C, _C), lambda i: (0, 0)),
        pl.BlockSpec((1, _C), lambda i: (0, 0)),
    ]


_tc1 = pl.pallas_call(
    _tc1_body,
    out_shape=jax.ShapeDtypeStruct((_NP, _C), jnp.float32),
    grid=(_GRID,),
    in_specs=_row_specs(),
    out_specs=pl.BlockSpec((_BLK, _C), lambda i: (i, 0)),
)

_tc2 = pl.pallas_call(
    _tc2_body,
    out_shape=jax.ShapeDtypeStruct((_NP, _OUT), jnp.float32),
    grid=(_GRID,),
    in_specs=_row_specs() + [
        pl.BlockSpec((_C, _OUT), lambda i: (0, 0)),
        pl.BlockSpec((1, _OUT), lambda i: (0, 0)),
    ],
    out_specs=pl.BlockSpec((_BLK, _OUT), lambda i: (i, 0)),
)


def kernel(feat_table, node_idx, edge_index,
           W_root1, W_nbr1, b1, W_root2, W_nbr2, b2, W_head, b_head):
    nidx = jnp.concatenate(
        [node_idx, jnp.zeros((_NP - _N,), jnp.int32)])
    src = jnp.concatenate(
        [edge_index[0], jnp.zeros((_EP - _E,), jnp.int32)])
    dst = jnp.concatenate(
        [edge_index[1], jnp.full((_EP - _E,), _N, jnp.int32)])

    x, deg = _get_gather_x_deg()(feat_table, nidx, dst)
    acc1 = _get_agg()(src, dst, x)
    h = _tc1(x, acc1, acc1, deg, deg, W_root1, W_nbr1, b1.reshape(1, _C))
    acc2 = _get_agg()(src, dst, h)
    out = _tc2(h, acc2, acc2, deg, deg, W_root2, W_nbr2, b2.reshape(1, _C),
               W_head, b_head.reshape(1, _OUT))
    return out[:_N]
